# baseline (speedup 1.0000x reference)
import functools

import jax
import jax.numpy as jnp
from jax import lax
from jax.experimental import pallas as pl
from jax.experimental.pallas import tpu as pltpu
from jax.experimental.pallas import tpu_sc as plsc

N = 10000
E = 320000
D = 128
G = 16

NP = 10240
EP = 327680
CHUNK = 64
NCHUNKS = EP // CHUNK
SUPER = 8
SLOTS = 5
NG0 = 34
NG1 = NCHUNKS // 16 // SUPER - NG0
CPT0 = NG0 * SUPER
CPT1 = NG1 * SUPER
ROWS_PER_TILE = NP // 16
RB_CHUNKS = ROWS_PER_TILE // CHUNK


def _sc_scatter_body(h_hbm, src_hbm, dst_hbm, out0_hbm, out1_hbm,
                     acc, sidx, didx, iidx, rows, gsem, ssem, isem_s, isem_d):
    c = lax.axis_index("c")
    s = lax.axis_index("s")
    r0 = s * ROWS_PER_TILE
    tile_base = jnp.where(c == 0, s * CPT0, 16 * CPT0 + s * CPT1)
    ngrp = jnp.where(c == 0, NG0, NG1)

    @pl.when(c == 0)
    def _():
        pltpu.sync_copy(h_hbm.at[pl.ds(r0, ROWS_PER_TILE)],
                        acc.at[pl.ds(r0, ROWS_PER_TILE)])

    @pl.when(c == 1)
    def _():
        zrow = jnp.zeros((16,), jnp.float32)

        def zfill(i, carry):
            for j in range(D // 16):
                rows[0, i, pl.ds(j * 16, 16)] = zrow
            return carry

        lax.fori_loop(0, CHUNK, zfill, 0)
        for k in range(RB_CHUNKS):
            pltpu.sync_copy(rows.at[0], acc.at[pl.ds(r0 + k * CHUNK, CHUNK)])

    plsc.subcore_barrier()

    def load_idx(g, buf):
        row0 = tile_base + g * SUPER
        pltpu.async_copy(src_hbm.at[pl.ds(row0, SUPER)], sidx.at[buf],
                         isem_s.at[buf])
        pltpu.async_copy(dst_hbm.at[pl.ds(row0, SUPER)], didx.at[buf],
                         isem_d.at[buf])

    def gather(buf, k):
        return pltpu.async_copy(h_hbm.at[sidx.at[buf, k]],
                                rows.at[k % SLOTS], gsem.at[k % SLOTS])

    def scatter(buf, k):
        return pltpu.async_copy(rows.at[k % SLOTS], acc.at[didx.at[buf, k]],
                                ssem.at[k % SLOTS], add=True)

    load_idx(0, 0)

    def group(g, carry):
        buf = lax.rem(g, 2)
        pltpu.make_async_copy(src_hbm.at[pl.ds(tile_base, SUPER)],
                              sidx.at[buf], isem_s.at[buf]).wait()
        pltpu.make_async_copy(dst_hbm.at[pl.ds(tile_base, SUPER)],
                              didx.at[buf], isem_d.at[buf]).wait()

        @pl.when(g + 1 < ngrp)
        def _():
            load_idx(g + 1, 1 - buf)

        gathers = [gather(buf, k) for k in range(SLOTS - 1)]
        scatters = [None] * SUPER
        for k in range(SUPER):
            gathers[k].wait()
            scatters[k] = scatter(buf, k)
            nxt = k + SLOTS - 1
            if nxt < SUPER:
                if k >= 1:
                    scatters[k - 1].wait()
                gathers.append(gather(buf, nxt))
        for k in range(SUPER - SLOTS, SUPER):
            scatters[k].wait()
        return carry

    lax.fori_loop(0, ngrp, group, 0)
    plsc.subcore_barrier()

    @pl.when(c == 0)
    def _():
        pltpu.sync_copy(acc.at[pl.ds(r0, ROWS_PER_TILE)],
                        out0_hbm.at[pl.ds(r0, ROWS_PER_TILE)])

    @pl.when(c == 1)
    def _():
        iota16 = lax.iota(jnp.int32, 16)
        for k in range(RB_CHUNKS):
            for q in range(CHUNK // 16):
                iidx[k, pl.ds(q * 16, 16)] = iota16 + (r0 + k * CHUNK + q * 16)
        scs = [None] * RB_CHUNKS
        for k in range(RB_CHUNKS):
            b = k % 2
            if k >= 2:
                scs[k - 2].wait()
            pltpu.sync_copy(acc.at[pl.ds(r0 + k * CHUNK, CHUNK)], rows.at[b])
            scs[k] = pltpu.async_copy(rows.at[b], out1_hbm.at[iidx.at[k]],
                                      ssem.at[b])
        for k in range(max(RB_CHUNKS - 2, 0), RB_CHUNKS):
            scs[k].wait()


@functools.lru_cache(maxsize=None)
def _get_sc_scatter():
    return pl.kernel(
        _sc_scatter_body,
        out_type=[jax.ShapeDtypeStruct((NP, D), jnp.float32),
                  jax.ShapeDtypeStruct((NP, D), jnp.float32)],
        mesh=plsc.VectorSubcoreMesh(core_axis_name="c", subcore_axis_name="s"),
        scratch_types=[
            pltpu.VMEM_SHARED((NP, D), jnp.float32),
            pltpu.VMEM((2, SUPER, CHUNK), jnp.int32),
            pltpu.VMEM((2, SUPER, CHUNK), jnp.int32),
            pltpu.VMEM((RB_CHUNKS, CHUNK), jnp.int32),
            pltpu.VMEM((SLOTS, CHUNK, D), jnp.float32),
            pltpu.SemaphoreType.DMA((SLOTS,)),
            pltpu.SemaphoreType.DMA((SLOTS,)),
            pltpu.SemaphoreType.DMA((2,)),
            pltpu.SemaphoreType.DMA((2,)),
        ],
    )


def _sc_scatter(h, src_p, dst_p):
    return _get_sc_scatter()(h, src_p, dst_p)

ROWS_BLK = 1280
NBLK = NP // ROWS_BLK


def _tc_mlp_body(p0_ref, p1_ref, w1_ref, b1_ref, w2_ref, b2_ref, o_ref, *,
                 final_relu):
    m = p0_ref[...] + p1_ref[...]
    t = jnp.maximum(jnp.dot(m, w1_ref[...],
                            preferred_element_type=jnp.float32) + b1_ref[...], 0.0)
    o = jnp.dot(t, w2_ref[...], preferred_element_type=jnp.float32) + b2_ref[...]
    if final_relu:
        o = jnp.maximum(o, 0.0)
    o_ref[...] = o


def _tc_mlp(p0, p1, w1, b1, w2, b2, final_relu):
    return pl.pallas_call(
        functools.partial(_tc_mlp_body, final_relu=final_relu),
        grid=(NBLK,),
        in_specs=[
            pl.BlockSpec((ROWS_BLK, D), lambda i: (i, 0)),
            pl.BlockSpec((ROWS_BLK, D), lambda i: (i, 0)),
            pl.BlockSpec((D, D), lambda i: (0, 0)),
            pl.BlockSpec((1, D), lambda i: (0, 0)),
            pl.BlockSpec((D, D), lambda i: (0, 0)),
            pl.BlockSpec((1, D), lambda i: (0, 0)),
        ],
        out_specs=pl.BlockSpec((ROWS_BLK, D), lambda i: (i, 0)),
        out_shape=jax.ShapeDtypeStruct((NP, D), jnp.float32),
    )(p0, p1, w1, b1, w2, b2)


def _tc_mlp_pool_body(bounds_ref, p0_ref, p1_ref, w1_ref, b1_ref, w2_ref,
                      b2_ref, o_ref):
    m = p0_ref[...] + p1_ref[...]
    t = jnp.maximum(jnp.dot(m, w1_ref[...],
                            preferred_element_type=jnp.float32) + b1_ref[...], 0.0)
    o = jnp.dot(t, w2_ref[...], preferred_element_type=jnp.float32) + b2_ref[...]

    @pl.when(pl.program_id(0) == 0)
    def _():
        o_ref[...] = jnp.full((G, D), -jnp.inf, jnp.float32)

    row0 = pl.program_id(0) * ROWS_BLK
    r2 = jax.lax.broadcasted_iota(jnp.int32, (ROWS_BLK, D), 0) + row0
    neg = jnp.full_like(o, -jnp.inf)
    segs = [jnp.max(jnp.where((r2 >= bounds_ref[g]) & (r2 < bounds_ref[g + 1]),
                              o, neg), axis=0)
            for g in range(G)]
    o_ref[...] = jnp.maximum(o_ref[...], jnp.stack(segs))


def _tc_mlp_pool(p0, p1, w1, b1, w2, b2, bounds):
    return pl.pallas_call(
        _tc_mlp_pool_body,
        grid=(NBLK,),
        in_specs=[
            pl.BlockSpec(memory_space=pltpu.MemorySpace.SMEM),
            pl.BlockSpec((ROWS_BLK, D), lambda i: (i, 0)),
            pl.BlockSpec((ROWS_BLK, D), lambda i: (i, 0)),
            pl.BlockSpec((D, D), lambda i: (0, 0)),
            pl.BlockSpec((1, D), lambda i: (0, 0)),
            pl.BlockSpec((D, D), lambda i: (0, 0)),
            pl.BlockSpec((1, D), lambda i: (0, 0)),
        ],
        out_specs=pl.BlockSpec((G, D), lambda i: (0, 0)),
        out_shape=jax.ShapeDtypeStruct((G, D), jnp.float32),
    )(bounds, p0, p1, w1, b1, w2, b2)


def kernel(x, edge_index, batch, W1_0, b1_0, W2_0, b2_0, W1_1, b1_1, W2_1,
           b2_1, W1_2, b1_2, W2_2, b2_2):
    src = edge_index[0]
    dst = edge_index[1]
    pad_e = EP - E
    src_p = jnp.concatenate([src, jnp.zeros((pad_e,), jnp.int32)]).reshape(NCHUNKS, CHUNK)
    dst_pad = N + (jnp.arange(pad_e, dtype=jnp.int32) % (NP - N))
    dst_p = jnp.concatenate([dst, dst_pad]).reshape(NCHUNKS, CHUNK)
    x_p = jnp.concatenate([x, jnp.zeros((NP - N, D), jnp.float32)])
    bounds = jnp.searchsorted(batch, jnp.arange(G + 1, dtype=jnp.int32)
                              ).astype(jnp.int32)
    b1s = [b1_0.reshape(1, D), b1_1.reshape(1, D), b1_2.reshape(1, D)]
    b2s = [b2_0.reshape(1, D), b2_1.reshape(1, D), b2_2.reshape(1, D)]
    w1s = [W1_0, W1_1, W1_2]
    w2s = [W2_0, W2_1, W2_2]

    h = x_p
    for i in range(2):
        p0, p1 = _sc_scatter(h, src_p, dst_p)
        h = _tc_mlp(p0, p1, w1s[i], b1s[i], w2s[i], b2s[i], final_relu=True)
    p0, p1 = _sc_scatter(h, src_p, dst_p)
    return _tc_mlp_pool(p0, p1, w1s[2], b1s[2], w2s[2], b2s[2], bounds)


# speedup vs baseline: 1.0020x; 1.0020x over previous
import functools

import jax
import jax.numpy as jnp
from jax import lax
from jax.experimental import pallas as pl
from jax.experimental.pallas import tpu as pltpu
from jax.experimental.pallas import tpu_sc as plsc

N = 10000
E = 320000
D = 128
G = 16

NP = 10240
EP = 327680
CHUNK = 64
NCHUNKS = EP // CHUNK
SUPER = 8
SLOTS = 4
NG0 = 34
NG1 = NCHUNKS // 16 // SUPER - NG0
CPT0 = NG0 * SUPER
CPT1 = NG1 * SUPER
ROWS_PER_TILE = NP // 16
RB_CHUNKS = ROWS_PER_TILE // CHUNK


def _sc_scatter_body(h_hbm, src_hbm, dst_hbm, out0_hbm, out1_hbm,
                     acc, sidx, didx, iidx, rows, gsem, ssem, isem_s, isem_d):
    c = lax.axis_index("c")
    s = lax.axis_index("s")
    r0 = s * ROWS_PER_TILE
    tile_base = jnp.where(c == 0, s * CPT0, 16 * CPT0 + s * CPT1)
    ngrp = jnp.where(c == 0, NG0, NG1)

    @pl.when(c == 0)
    def _():
        pltpu.sync_copy(h_hbm.at[pl.ds(r0, ROWS_PER_TILE)],
                        acc.at[pl.ds(r0, ROWS_PER_TILE)])

    @pl.when(c == 1)
    def _():
        zrow = jnp.zeros((16,), jnp.float32)

        def zfill(i, carry):
            for j in range(D // 16):
                rows[0, i, pl.ds(j * 16, 16)] = zrow
            return carry

        lax.fori_loop(0, CHUNK, zfill, 0)
        for k in range(RB_CHUNKS):
            pltpu.sync_copy(rows.at[0], acc.at[pl.ds(r0 + k * CHUNK, CHUNK)])

    plsc.subcore_barrier()

    def load_idx(g, buf):
        row0 = tile_base + g * SUPER
        pltpu.async_copy(src_hbm.at[pl.ds(row0, SUPER)], sidx.at[buf],
                         isem_s.at[buf])
        pltpu.async_copy(dst_hbm.at[pl.ds(row0, SUPER)], didx.at[buf],
                         isem_d.at[buf])

    def gather(buf, k):
        return pltpu.async_copy(h_hbm.at[sidx.at[buf, k]],
                                rows.at[k % SLOTS], gsem.at[k % SLOTS])

    def scatter(buf, k):
        return pltpu.async_copy(rows.at[k % SLOTS], acc.at[didx.at[buf, k]],
                                ssem.at[k % SLOTS], add=True)

    load_idx(0, 0)

    def group(g, carry):
        buf = lax.rem(g, 2)
        pltpu.make_async_copy(src_hbm.at[pl.ds(tile_base, SUPER)],
                              sidx.at[buf], isem_s.at[buf]).wait()
        pltpu.make_async_copy(dst_hbm.at[pl.ds(tile_base, SUPER)],
                              didx.at[buf], isem_d.at[buf]).wait()

        @pl.when(g + 1 < ngrp)
        def _():
            load_idx(g + 1, 1 - buf)

        gathers = [gather(buf, k) for k in range(SLOTS - 1)]
        scatters = [None] * SUPER
        for k in range(SUPER):
            gathers[k].wait()
            scatters[k] = scatter(buf, k)
            nxt = k + SLOTS - 1
            if nxt < SUPER:
                if k >= 1:
                    scatters[k - 1].wait()
                gathers.append(gather(buf, nxt))
        for k in range(SUPER - SLOTS, SUPER):
            scatters[k].wait()
        return carry

    lax.fori_loop(0, ngrp, group, 0)
    plsc.subcore_barrier()

    @pl.when(c == 0)
    def _():
        pltpu.sync_copy(acc.at[pl.ds(r0, ROWS_PER_TILE)],
                        out0_hbm.at[pl.ds(r0, ROWS_PER_TILE)])

    @pl.when(c == 1)
    def _():
        iota16 = lax.iota(jnp.int32, 16)
        for k in range(RB_CHUNKS):
            for q in range(CHUNK // 16):
                iidx[k, pl.ds(q * 16, 16)] = iota16 + (r0 + k * CHUNK + q * 16)
        scs = [None] * RB_CHUNKS
        for k in range(RB_CHUNKS):
            b = k % 2
            if k >= 2:
                scs[k - 2].wait()
            pltpu.sync_copy(acc.at[pl.ds(r0 + k * CHUNK, CHUNK)], rows.at[b])
            scs[k] = pltpu.async_copy(rows.at[b], out1_hbm.at[iidx.at[k]],
                                      ssem.at[b])
        for k in range(max(RB_CHUNKS - 2, 0), RB_CHUNKS):
            scs[k].wait()


@functools.lru_cache(maxsize=None)
def _get_sc_scatter():
    return pl.kernel(
        _sc_scatter_body,
        out_type=[jax.ShapeDtypeStruct((NP, D), jnp.float32),
                  jax.ShapeDtypeStruct((NP, D), jnp.float32)],
        mesh=plsc.VectorSubcoreMesh(core_axis_name="c", subcore_axis_name="s"),
        scratch_types=[
            pltpu.VMEM_SHARED((NP, D), jnp.float32),
            pltpu.VMEM((2, SUPER, CHUNK), jnp.int32),
            pltpu.VMEM((2, SUPER, CHUNK), jnp.int32),
            pltpu.VMEM((RB_CHUNKS, CHUNK), jnp.int32),
            pltpu.VMEM((SLOTS, CHUNK, D), jnp.float32),
            pltpu.SemaphoreType.DMA((SLOTS,)),
            pltpu.SemaphoreType.DMA((SLOTS,)),
            pltpu.SemaphoreType.DMA((2,)),
            pltpu.SemaphoreType.DMA((2,)),
        ],
    )


def _sc_scatter(h, src_p, dst_p):
    return _get_sc_scatter()(h, src_p, dst_p)

ROWS_BLK = 1280
NBLK = NP // ROWS_BLK


def _tc_mlp_body(p0_ref, p1_ref, w1_ref, b1_ref, w2_ref, b2_ref, o_ref, *,
                 final_relu):
    m = p0_ref[...] + p1_ref[...]
    t = jnp.maximum(jnp.dot(m, w1_ref[...],
                            preferred_element_type=jnp.float32) + b1_ref[...], 0.0)
    o = jnp.dot(t, w2_ref[...], preferred_element_type=jnp.float32) + b2_ref[...]
    if final_relu:
        o = jnp.maximum(o, 0.0)
    o_ref[...] = o


def _tc_mlp(p0, p1, w1, b1, w2, b2, final_relu):
    return pl.pallas_call(
        functools.partial(_tc_mlp_body, final_relu=final_relu),
        grid=(NBLK,),
        in_specs=[
            pl.BlockSpec((ROWS_BLK, D), lambda i: (i, 0)),
            pl.BlockSpec((ROWS_BLK, D), lambda i: (i, 0)),
            pl.BlockSpec((D, D), lambda i: (0, 0)),
            pl.BlockSpec((1, D), lambda i: (0, 0)),
            pl.BlockSpec((D, D), lambda i: (0, 0)),
            pl.BlockSpec((1, D), lambda i: (0, 0)),
        ],
        out_specs=pl.BlockSpec((ROWS_BLK, D), lambda i: (i, 0)),
        out_shape=jax.ShapeDtypeStruct((NP, D), jnp.float32),
    )(p0, p1, w1, b1, w2, b2)


def _tc_mlp_pool_body(bounds_ref, p0_ref, p1_ref, w1_ref, b1_ref, w2_ref,
                      b2_ref, o_ref):
    m = p0_ref[...] + p1_ref[...]
    t = jnp.maximum(jnp.dot(m, w1_ref[...],
                            preferred_element_type=jnp.float32) + b1_ref[...], 0.0)
    o = jnp.dot(t, w2_ref[...], preferred_element_type=jnp.float32) + b2_ref[...]

    @pl.when(pl.program_id(0) == 0)
    def _():
        o_ref[...] = jnp.full((G, D), -jnp.inf, jnp.float32)

    row0 = pl.program_id(0) * ROWS_BLK
    r2 = jax.lax.broadcasted_iota(jnp.int32, (ROWS_BLK, D), 0) + row0
    neg = jnp.full_like(o, -jnp.inf)
    segs = [jnp.max(jnp.where((r2 >= bounds_ref[g]) & (r2 < bounds_ref[g + 1]),
                              o, neg), axis=0)
            for g in range(G)]
    o_ref[...] = jnp.maximum(o_ref[...], jnp.stack(segs))


def _tc_mlp_pool(p0, p1, w1, b1, w2, b2, bounds):
    return pl.pallas_call(
        _tc_mlp_pool_body,
        grid=(NBLK,),
        in_specs=[
            pl.BlockSpec(memory_space=pltpu.MemorySpace.SMEM),
            pl.BlockSpec((ROWS_BLK, D), lambda i: (i, 0)),
            pl.BlockSpec((ROWS_BLK, D), lambda i: (i, 0)),
            pl.BlockSpec((D, D), lambda i: (0, 0)),
            pl.BlockSpec((1, D), lambda i: (0, 0)),
            pl.BlockSpec((D, D), lambda i: (0, 0)),
            pl.BlockSpec((1, D), lambda i: (0, 0)),
        ],
        out_specs=pl.BlockSpec((G, D), lambda i: (0, 0)),
        out_shape=jax.ShapeDtypeStruct((G, D), jnp.float32),
    )(bounds, p0, p1, w1, b1, w2, b2)


def kernel(x, edge_index, batch, W1_0, b1_0, W2_0, b2_0, W1_1, b1_1, W2_1,
           b2_1, W1_2, b1_2, W2_2, b2_2):
    src = edge_index[0]
    dst = edge_index[1]
    pad_e = EP - E
    src_p = jnp.concatenate([src, jnp.zeros((pad_e,), jnp.int32)]).reshape(NCHUNKS, CHUNK)
    dst_pad = N + (jnp.arange(pad_e, dtype=jnp.int32) % (NP - N))
    dst_p = jnp.concatenate([dst, dst_pad]).reshape(NCHUNKS, CHUNK)
    x_p = jnp.concatenate([x, jnp.zeros((NP - N, D), jnp.float32)])
    bounds = jnp.searchsorted(batch, jnp.arange(G + 1, dtype=jnp.int32)
                              ).astype(jnp.int32)
    b1s = [b1_0.reshape(1, D), b1_1.reshape(1, D), b1_2.reshape(1, D)]
    b2s = [b2_0.reshape(1, D), b2_1.reshape(1, D), b2_2.reshape(1, D)]
    w1s = [W1_0, W1_1, W1_2]
    w2s = [W2_0, W2_1, W2_2]

    h = x_p
    for i in range(2):
        p0, p1 = _sc_scatter(h, src_p, dst_p)
        h = _tc_mlp(p0, p1, w1s[i], b1s[i], w2s[i], b2s[i], final_relu=True)
    p0, p1 = _sc_scatter(h, src_p, dst_p)
    return _tc_mlp_pool(p0, p1, w1s[2], b1s[2], w2s[2], b2s[2], bounds)
